# Initial kernel scaffold; baseline (speedup 1.0000x reference)
import jax
import jax.numpy as jnp
from jax.experimental import pallas as pl


def kernel(h, Wq_comp, Wq_up, Wk, Wv, qn_w, kn_w, Wg, bg, sink, Wo1, Wo2):
    raise NotImplementedError("write your pallas kernel here")


# speedup vs baseline: 1.1023x; 1.1023x over previous
import math

import jax
import jax.numpy as jnp
from jax.experimental import pallas as pl
from jax.experimental.pallas import tpu as pltpu

T = 2048
HID = 768
H = 12
D = 64
QCD = 384
ROPE = 32
HALF = ROPE // 2
THETA = 10000.0
WIN = 256
CR = 32
SELK = 16
G = 2
INTER = 1024

TQ = 256
NT = T // TQ
NB = T // CR
SCALE = 1.0 / math.sqrt(D)
NEG = -1e30
F32 = jnp.float32


def _iota(shape, dim):
    return jax.lax.broadcasted_iota(jnp.int32, shape, dim)


def _rope_rms(x, cos, sin, w):
    x1 = x[:, :HALF]
    x2 = x[:, HALF:ROPE]
    rot = jnp.concatenate([x1 * cos - x2 * sin, x1 * sin + x2 * cos], axis=1)
    x = jnp.concatenate([rot, x[:, ROPE:]], axis=1)
    ms = jnp.mean(x * x, axis=1, keepdims=True)
    return x * jax.lax.rsqrt(ms + 1e-6) * w


def _dot(a, b):
    return jnp.dot(a, b, preferred_element_type=F32,
                   precision=jax.lax.Precision.HIGHEST)


def _dotd(a, b):
    return jnp.dot(a, b, preferred_element_type=F32)


def _dotd_nt(a, b):
    return jax.lax.dot_general(a, b, (((1,), (1,)), ((), ())),
                               preferred_element_type=F32)


def _proj_body(h_ref, wqc_ref, wqu_ref, wk_ref, wv_ref, kn_ref, cs_ref, sn_ref,
               q_ref, k_ref, v_ref, kc_ref, vc_ref):
    hx = h_ref[...]
    cq = _dotd(hx, wqc_ref[...])
    qf = _dotd(cq, wqu_ref[...])
    kx = _dotd(hx, wk_ref[...])
    vx = _dotd(hx, wv_ref[...])
    kx = _rope_rms(kx, cs_ref[...], sn_ref[...], kn_ref[...])
    for hh in range(H):
        q_ref[hh] = qf[:, D * hh:D * (hh + 1)]
    k_ref[...] = kx
    v_ref[...] = vx
    rows = _iota((TQ // CR, TQ), 0)
    cols = _iota((TQ // CR, TQ), 1)
    avg = jnp.where(cols // CR == rows, 1.0 / CR, 0.0).astype(F32)
    kc_ref[...] = _dot(avg, kx)
    vc_ref[...] = _dot(avg, vx)


def _attn_body(q_ref, k_ref, v_ref, kc_ref, vc_ref, e_ref, wg_ref, bgp_ref,
               qn_ref, cs_ref, sn_ref, sink_ref, out_ref):
    h_id = pl.program_id(0)
    i = pl.program_id(1)
    q = _rope_rms(q_ref[0], cs_ref[...], sn_ref[...], qn_ref[...])
    qpos = i * TQ + _iota((TQ, 1), 0)

    kc = kc_ref[...]
    sblk = _dotd_nt(q, kc) * SCALE
    be = _iota((TQ, NB), 1) * CR + (CR - 1)
    valid = be <= qpos
    sblk_m = jnp.where(valid, sblk, NEG)
    anyb = qpos >= (CR - 1)

    sk = sink_ref[h_id]
    s_c = jnp.where(anyb, sblk_m, sblk)
    m_c = jnp.maximum(jnp.max(s_c, axis=1, keepdims=True), sk)
    e_c = jnp.exp(s_c - m_c)
    den_c = jnp.sum(e_c, axis=1, keepdims=True) + jnp.exp(sk - m_c)
    comp = _dotd(e_c, vc_ref[...]) / den_c
    comp = jnp.where(anyb, comp, 0.0)

    colid = _iota((TQ, NB), 1)
    s_work = sblk_m
    selb = jnp.zeros((TQ, NB), jnp.bool_)
    for _ in range(SELK):
        m = jnp.max(s_work, axis=1, keepdims=True)
        eqm = s_work == m
        fidx = jnp.min(jnp.where(eqm, colid, NB), axis=1, keepdims=True)
        first = colid == fidx
        selb = jnp.logical_or(selb, first)
        s_work = jnp.where(first, -jnp.inf, s_work)
    selb_f = selb.astype(F32)

    def body(c, carry):
        m, l, acc = carry
        kb = k_ref[pl.ds(c * TQ, TQ), :]
        s = _dotd_nt(q, kb) * SCALE
        eb = e_ref[:, pl.ds(c * TQ, TQ)]
        s = s + (_dotd(selb_f, eb) - 1.0) * 1e30
        jpos = c * TQ + _iota((TQ, TQ), 1)
        s = jnp.where(jpos <= qpos, s, NEG)
        mnew = jnp.maximum(m, jnp.max(s, axis=1, keepdims=True))
        p = jnp.exp(s - mnew)
        alpha = jnp.exp(m - mnew)
        l = l * alpha + jnp.sum(p, axis=1, keepdims=True)
        vb = v_ref[pl.ds(c * TQ, TQ), :]
        acc = acc * alpha + _dotd(p, vb)
        return mnew, l, acc

    m0 = jnp.full((TQ, 1), NEG, F32)
    l0 = jnp.zeros((TQ, 1), F32)
    a0 = jnp.zeros((TQ, D), F32)
    _, l_s, acc_s = jax.lax.fori_loop(0, i + 1, body, (m0, l0, a0))
    sel = acc_s / l_s

    prev = jnp.maximum(i - 1, 0)
    kb1 = k_ref[pl.ds(i * TQ, TQ), :]
    vb1 = v_ref[pl.ds(i * TQ, TQ), :]
    kb0 = k_ref[pl.ds(prev * TQ, TQ), :]
    vb0 = v_ref[pl.ds(prev * TQ, TQ), :]
    s1 = _dotd_nt(q, kb1) * SCALE
    jpos1 = i * TQ + _iota((TQ, TQ), 1)
    s1 = jnp.where(jpos1 <= qpos, s1, NEG)
    s0 = _dotd_nt(q, kb0) * SCALE
    jpos0 = prev * TQ + _iota((TQ, TQ), 1)
    ok0 = jnp.logical_and(qpos - jpos0 < WIN, i > 0)
    s0 = jnp.where(ok0, s0, NEG)
    m_w = jnp.maximum(jnp.max(s1, axis=1, keepdims=True),
                      jnp.max(s0, axis=1, keepdims=True))
    m_w = jnp.maximum(m_w, sk)
    e0 = jnp.exp(s0 - m_w)
    e1 = jnp.exp(s1 - m_w)
    den_w = (jnp.sum(e0, axis=1, keepdims=True)
             + jnp.sum(e1, axis=1, keepdims=True) + jnp.exp(sk - m_w))
    sw = (_dotd(e0, vb0) + _dotd(e1, vb1)) / den_w

    g = jax.nn.sigmoid(_dotd(q, wg_ref[...]) + bgp_ref[...])
    gs = [g[:, j:j + 1] for j in range(3)]
    gsum = jnp.maximum(gs[0] + gs[1] + gs[2], 1e-6)
    out_ref[0] = (gs[0] * comp + gs[1] * sel + gs[2] * sw) / gsum


def _out_body(x_ref, w1_ref, w2_ref, o_ref):
    xs = [x_ref[hh] for hh in range(H)]
    x0 = jnp.concatenate(xs[:H // G], axis=1)
    x1 = jnp.concatenate(xs[H // G:], axis=1)
    y0 = _dotd(x0, w1_ref[0])
    y1 = _dotd(x1, w1_ref[1])
    o_ref[...] = _dotd(y0, w2_ref[:INTER, :]) + _dotd(y1, w2_ref[INTER:, :])


def kernel(h, Wq_comp, Wq_up, Wk, Wv, qn_w, kn_w, Wg, bg, sink, Wo1, Wo2):
    h2 = h[0]
    wqcT = Wq_comp.T
    wquT = Wq_up.T
    wkT = Wk.T
    wvT = Wv.T
    kn2 = kn_w[None, :]
    qn2 = qn_w[None, :]
    wg_pad = jnp.zeros((D, 8), F32).at[:, :3].set(Wg.T)
    bg_pad = jnp.zeros((1, 8), F32).at[0, :3].set(bg)
    e_mat = (jnp.arange(T)[None, :] // CR == jnp.arange(NB)[:, None]).astype(F32)
    pos = jnp.arange(T, dtype=F32)
    inv = 1.0 / (THETA ** (jnp.arange(0, ROPE, 2, dtype=F32) / ROPE))
    fr = pos[:, None] * inv[None, :]
    cs_t, sn_t = jnp.cos(fr), jnp.sin(fr)

    q, k, v, kc, vc = pl.pallas_call(
        _proj_body,
        grid=(NT,),
        in_specs=[
            pl.BlockSpec((TQ, HID), lambda i: (i, 0)),
            pl.BlockSpec((HID, QCD), lambda i: (0, 0)),
            pl.BlockSpec((QCD, HID), lambda i: (0, 0)),
            pl.BlockSpec((HID, D), lambda i: (0, 0)),
            pl.BlockSpec((HID, D), lambda i: (0, 0)),
            pl.BlockSpec((1, D), lambda i: (0, 0)),
            pl.BlockSpec((TQ, HALF), lambda i: (i, 0)),
            pl.BlockSpec((TQ, HALF), lambda i: (i, 0)),
        ],
        out_specs=[
            pl.BlockSpec((H, TQ, D), lambda i: (0, i, 0)),
            pl.BlockSpec((TQ, D), lambda i: (i, 0)),
            pl.BlockSpec((TQ, D), lambda i: (i, 0)),
            pl.BlockSpec((TQ // CR, D), lambda i: (i, 0)),
            pl.BlockSpec((TQ // CR, D), lambda i: (i, 0)),
        ],
        out_shape=[
            jax.ShapeDtypeStruct((H, T, D), F32),
            jax.ShapeDtypeStruct((T, D), F32),
            jax.ShapeDtypeStruct((T, D), F32),
            jax.ShapeDtypeStruct((NB, D), F32),
            jax.ShapeDtypeStruct((NB, D), F32),
        ],
    )(h2, wqcT, wquT, wkT, wvT, kn2, cs_t, sn_t)

    mixed = pl.pallas_call(
        _attn_body,
        grid=(H, NT),
        in_specs=[
            pl.BlockSpec((1, TQ, D), lambda hh, i: (hh, i, 0)),
            pl.BlockSpec((T, D), lambda hh, i: (0, 0)),
            pl.BlockSpec((T, D), lambda hh, i: (0, 0)),
            pl.BlockSpec((NB, D), lambda hh, i: (0, 0)),
            pl.BlockSpec((NB, D), lambda hh, i: (0, 0)),
            pl.BlockSpec((NB, T), lambda hh, i: (0, 0)),
            pl.BlockSpec((D, 8), lambda hh, i: (0, 0)),
            pl.BlockSpec((1, 8), lambda hh, i: (0, 0)),
            pl.BlockSpec((1, D), lambda hh, i: (0, 0)),
            pl.BlockSpec((TQ, HALF), lambda hh, i: (i, 0)),
            pl.BlockSpec((TQ, HALF), lambda hh, i: (i, 0)),
            pl.BlockSpec(memory_space=pltpu.SMEM),
        ],
        out_specs=pl.BlockSpec((1, TQ, D), lambda hh, i: (hh, i, 0)),
        out_shape=jax.ShapeDtypeStruct((H, T, D), F32),
    )(q, k, v, kc, vc, e_mat, wg_pad, bg_pad, qn2, cs_t, sn_t, sink)

    w1t = jnp.transpose(Wo1, (0, 2, 1))
    w2t = Wo2.T
    y = pl.pallas_call(
        _out_body,
        grid=(NT,),
        in_specs=[
            pl.BlockSpec((H, TQ, D), lambda i: (0, i, 0)),
            pl.BlockSpec((G, QCD, INTER), lambda i: (0, 0, 0)),
            pl.BlockSpec((G * INTER, HID), lambda i: (0, 0)),
        ],
        out_specs=pl.BlockSpec((TQ, HID), lambda i: (i, 0)),
        out_shape=jax.ShapeDtypeStruct((T, HID), F32),
    )(mixed, w1t, w2t)
    return y[None]


# speedup vs baseline: 1.7040x; 1.5459x over previous
import math

import jax
import jax.numpy as jnp
from jax.experimental import pallas as pl
from jax.experimental.pallas import tpu as pltpu

T = 2048
HID = 768
H = 12
D = 64
QCD = 384
ROPE = 32
HALF = ROPE // 2
THETA = 10000.0
WIN = 256
CR = 32
SELK = 16
G = 2
INTER = 1024

TQ = 256
NT = T // TQ
NB = T // CR
SCALE = 1.0 / math.sqrt(D)
NEG = -1e30
F32 = jnp.float32


def _iota(shape, dim):
    return jax.lax.broadcasted_iota(jnp.int32, shape, dim)


def _rope_rms(x, cos, sin, w):
    x1 = x[:, :HALF]
    x2 = x[:, HALF:ROPE]
    rot = jnp.concatenate([x1 * cos - x2 * sin, x1 * sin + x2 * cos], axis=1)
    x = jnp.concatenate([rot, x[:, ROPE:]], axis=1)
    ms = jnp.mean(x * x, axis=1, keepdims=True)
    return x * jax.lax.rsqrt(ms + 1e-6) * w


def _dot(a, b):
    return jnp.dot(a, b, preferred_element_type=F32,
                   precision=jax.lax.Precision.HIGHEST)


def _dotd(a, b):
    return jnp.dot(a, b, preferred_element_type=F32)


def _dotd_nt(a, b):
    return jax.lax.dot_general(a, b, (((1,), (1,)), ((), ())),
                               preferred_element_type=F32)


def _proj_body(h_ref, wqc_ref, wqu_ref, wk_ref, wv_ref, kn_ref, cs_ref, sn_ref,
               avg_ref, q_ref, k_ref, v_ref, kc_ref, vc_ref):
    hx = h_ref[...]
    cq = _dotd(hx, wqc_ref[...])
    qf = _dotd(cq, wqu_ref[...])
    kx = _dotd(hx, wk_ref[...])
    vx = _dotd(hx, wv_ref[...])
    kx = _rope_rms(kx, cs_ref[...], sn_ref[...], kn_ref[...])
    for hh in range(H):
        q_ref[hh] = qf[:, D * hh:D * (hh + 1)]
    k_ref[...] = kx
    v_ref[...] = vx
    avg = avg_ref[...]
    kc_ref[...] = _dot(avg, kx)
    vc_ref[...] = _dot(avg, vx)


def _attn_body(q_ref, k_ref, v_ref, kc_ref, vc_ref, e_ref, wg_ref, bgp_ref,
               qn_ref, cs_ref, sn_ref, sink_ref, out_ref):
    h_id = pl.program_id(0)
    i = pl.program_id(1)
    q = _rope_rms(q_ref[0], cs_ref[...], sn_ref[...], qn_ref[...])
    qpos = i * TQ + _iota((TQ, 1), 0)

    kc = kc_ref[...]
    sblk = _dotd_nt(q, kc) * SCALE
    be = _iota((TQ, NB), 1) * CR + (CR - 1)
    valid = be <= qpos
    sblk_m = jnp.where(valid, sblk, NEG)
    anyb = qpos >= (CR - 1)

    sk = sink_ref[h_id]
    s_c = jnp.where(anyb, sblk_m, sblk)
    m_c = jnp.maximum(jnp.max(s_c, axis=1, keepdims=True), sk)
    e_c = jnp.exp(s_c - m_c)
    den_c = jnp.sum(e_c, axis=1, keepdims=True) + jnp.exp(sk - m_c)
    comp = _dotd(e_c, vc_ref[...]) / den_c
    comp = jnp.where(anyb, comp, 0.0)

    colid = _iota((TQ, NB), 1)
    low = (qpos + 1) // CR < SELK
    s_work = sblk_m
    selb = jnp.zeros((TQ, NB), jnp.bool_)
    for _ in range(SELK):
        m = jnp.max(s_work, axis=1, keepdims=True)
        eqm = jnp.logical_and(s_work == m, m > -1e29)
        selb = jnp.logical_or(selb, eqm)
        s_work = jnp.where(eqm, -jnp.inf, s_work)
    selb = jnp.logical_or(jnp.logical_and(low, colid < SELK),
                          jnp.logical_and(jnp.logical_not(low), selb))
    selb_f = selb.astype(F32)

    def chunk(c, carry, diag):
        m, l, acc = carry
        kb = k_ref[pl.ds(c * TQ, TQ), :]
        s = _dotd_nt(q, kb) * SCALE
        eb = e_ref[:, pl.ds(c * TQ, TQ)]
        s = s + (_dotd(selb_f, eb) - 1.0) * 1e30
        if diag:
            jpos = c * TQ + _iota((TQ, TQ), 1)
            s = jnp.where(jpos <= qpos, s, NEG)
        mnew = jnp.maximum(m, jnp.max(s, axis=1, keepdims=True))
        p = jnp.exp(s - mnew)
        alpha = jnp.exp(m - mnew)
        l = l * alpha + jnp.sum(p, axis=1, keepdims=True)
        vb = v_ref[pl.ds(c * TQ, TQ), :]
        acc = acc * alpha + _dotd(p, vb)
        return mnew, l, acc

    m0 = jnp.full((TQ, 1), NEG, F32)
    l0 = jnp.zeros((TQ, 1), F32)
    a0 = jnp.zeros((TQ, D), F32)
    carry = jax.lax.fori_loop(0, i, lambda c, cr: chunk(c, cr, False),
                              (m0, l0, a0))
    _, l_s, acc_s = chunk(i, carry, True)
    sel = acc_s / l_s

    prev = jnp.maximum(i - 1, 0)
    kb1 = k_ref[pl.ds(i * TQ, TQ), :]
    vb1 = v_ref[pl.ds(i * TQ, TQ), :]
    kb0 = k_ref[pl.ds(prev * TQ, TQ), :]
    vb0 = v_ref[pl.ds(prev * TQ, TQ), :]
    s1 = _dotd_nt(q, kb1) * SCALE
    jpos1 = i * TQ + _iota((TQ, TQ), 1)
    s1 = jnp.where(jpos1 <= qpos, s1, NEG)
    s0 = _dotd_nt(q, kb0) * SCALE
    jpos0 = prev * TQ + _iota((TQ, TQ), 1)
    ok0 = jnp.logical_and(qpos - jpos0 < WIN, i > 0)
    s0 = jnp.where(ok0, s0, NEG)
    m_w = jnp.maximum(jnp.max(s1, axis=1, keepdims=True),
                      jnp.max(s0, axis=1, keepdims=True))
    m_w = jnp.maximum(m_w, sk)
    e0 = jnp.exp(s0 - m_w)
    e1 = jnp.exp(s1 - m_w)
    den_w = (jnp.sum(e0, axis=1, keepdims=True)
             + jnp.sum(e1, axis=1, keepdims=True) + jnp.exp(sk - m_w))
    sw = (_dotd(e0, vb0) + _dotd(e1, vb1)) / den_w

    g = jax.nn.sigmoid(_dotd(q, wg_ref[...]) + bgp_ref[...])
    gs = [g[:, j:j + 1] for j in range(3)]
    gsum = jnp.maximum(gs[0] + gs[1] + gs[2], 1e-6)
    out_ref[0] = (gs[0] * comp + gs[1] * sel + gs[2] * sw) / gsum


def _out_body(x_ref, w1_ref, w2_ref, o_ref):
    xs = [x_ref[hh] for hh in range(H)]
    x0 = jnp.concatenate(xs[:H // G], axis=1)
    x1 = jnp.concatenate(xs[H // G:], axis=1)
    y0 = _dotd(x0, w1_ref[0])
    y1 = _dotd(x1, w1_ref[1])
    o_ref[...] = _dotd(y0, w2_ref[:INTER, :]) + _dotd(y1, w2_ref[INTER:, :])


def kernel(h, Wq_comp, Wq_up, Wk, Wv, qn_w, kn_w, Wg, bg, sink, Wo1, Wo2):
    h2 = h[0]
    wqcT = Wq_comp.T
    wquT = Wq_up.T
    wkT = Wk.T
    wvT = Wv.T
    kn2 = kn_w[None, :]
    qn2 = qn_w[None, :]
    wg_pad = jnp.zeros((D, 8), F32).at[:, :3].set(Wg.T)
    bg_pad = jnp.zeros((1, 8), F32).at[0, :3].set(bg)
    e_mat = (jnp.arange(T)[None, :] // CR == jnp.arange(NB)[:, None]).astype(F32)
    pos = jnp.arange(T, dtype=F32)
    inv = 1.0 / (THETA ** (jnp.arange(0, ROPE, 2, dtype=F32) / ROPE))
    fr = pos[:, None] * inv[None, :]
    cs_t, sn_t = jnp.cos(fr), jnp.sin(fr)
    avg_m = ((jnp.arange(TQ)[None, :] // CR == jnp.arange(TQ // CR)[:, None])
             .astype(F32) / CR)

    q, k, v, kc, vc = pl.pallas_call(
        _proj_body,
        grid=(NT,),
        in_specs=[
            pl.BlockSpec((TQ, HID), lambda i: (i, 0)),
            pl.BlockSpec((HID, QCD), lambda i: (0, 0)),
            pl.BlockSpec((QCD, HID), lambda i: (0, 0)),
            pl.BlockSpec((HID, D), lambda i: (0, 0)),
            pl.BlockSpec((HID, D), lambda i: (0, 0)),
            pl.BlockSpec((1, D), lambda i: (0, 0)),
            pl.BlockSpec((TQ, HALF), lambda i: (i, 0)),
            pl.BlockSpec((TQ, HALF), lambda i: (i, 0)),
            pl.BlockSpec((TQ // CR, TQ), lambda i: (0, 0)),
        ],
        out_specs=[
            pl.BlockSpec((H, TQ, D), lambda i: (0, i, 0)),
            pl.BlockSpec((TQ, D), lambda i: (i, 0)),
            pl.BlockSpec((TQ, D), lambda i: (i, 0)),
            pl.BlockSpec((TQ // CR, D), lambda i: (i, 0)),
            pl.BlockSpec((TQ // CR, D), lambda i: (i, 0)),
        ],
        out_shape=[
            jax.ShapeDtypeStruct((H, T, D), F32),
            jax.ShapeDtypeStruct((T, D), F32),
            jax.ShapeDtypeStruct((T, D), F32),
            jax.ShapeDtypeStruct((NB, D), F32),
            jax.ShapeDtypeStruct((NB, D), F32),
        ],
    )(h2, wqcT, wquT, wkT, wvT, kn2, cs_t, sn_t, avg_m)

    mixed = pl.pallas_call(
        _attn_body,
        grid=(H, NT),
        in_specs=[
            pl.BlockSpec((1, TQ, D), lambda hh, i: (hh, i, 0)),
            pl.BlockSpec((T, D), lambda hh, i: (0, 0)),
            pl.BlockSpec((T, D), lambda hh, i: (0, 0)),
            pl.BlockSpec((NB, D), lambda hh, i: (0, 0)),
            pl.BlockSpec((NB, D), lambda hh, i: (0, 0)),
            pl.BlockSpec((NB, T), lambda hh, i: (0, 0)),
            pl.BlockSpec((D, 8), lambda hh, i: (0, 0)),
            pl.BlockSpec((1, 8), lambda hh, i: (0, 0)),
            pl.BlockSpec((1, D), lambda hh, i: (0, 0)),
            pl.BlockSpec((TQ, HALF), lambda hh, i: (i, 0)),
            pl.BlockSpec((TQ, HALF), lambda hh, i: (i, 0)),
            pl.BlockSpec(memory_space=pltpu.SMEM),
        ],
        out_specs=pl.BlockSpec((1, TQ, D), lambda hh, i: (hh, i, 0)),
        out_shape=jax.ShapeDtypeStruct((H, T, D), F32),
    )(q, k, v, kc, vc, e_mat, wg_pad, bg_pad, qn2, cs_t, sn_t, sink)

    w1t = jnp.transpose(Wo1, (0, 2, 1))
    w2t = Wo2.T
    y = pl.pallas_call(
        _out_body,
        grid=(NT,),
        in_specs=[
            pl.BlockSpec((H, TQ, D), lambda i: (0, i, 0)),
            pl.BlockSpec((G, QCD, INTER), lambda i: (0, 0, 0)),
            pl.BlockSpec((G * INTER, HID), lambda i: (0, 0)),
        ],
        out_specs=pl.BlockSpec((TQ, HID), lambda i: (i, 0)),
        out_shape=jax.ShapeDtypeStruct((T, HID), F32),
    )(mixed, w1t, w2t)
    return y[None]


# speedup vs baseline: 2.1485x; 1.2609x over previous
import math

import jax
import jax.numpy as jnp
from jax.experimental import pallas as pl
from jax.experimental.pallas import tpu as pltpu

T = 2048
HID = 768
H = 12
D = 64
QCD = 384
ROPE = 32
HALF = ROPE // 2
THETA = 10000.0
WIN = 256
CR = 32
SELK = 16
G = 2
INTER = 1024

TQ = 256
NT = T // TQ
NB = T // CR
HB = 2
SCALE = 1.0 / math.sqrt(D)
NEG = -1e30
F32 = jnp.float32


def _iota(shape, dim):
    return jax.lax.broadcasted_iota(jnp.int32, shape, dim)


def _rope_rms(x, cos, sin, w):
    x1 = x[:, :HALF]
    x2 = x[:, HALF:ROPE]
    rot = jnp.concatenate([x1 * cos - x2 * sin, x1 * sin + x2 * cos], axis=1)
    x = jnp.concatenate([rot, x[:, ROPE:]], axis=1)
    ms = jnp.mean(x * x, axis=1, keepdims=True)
    return x * jax.lax.rsqrt(ms + 1e-6) * w


def _dot(a, b):
    return jnp.dot(a, b, preferred_element_type=F32,
                   precision=jax.lax.Precision.HIGHEST)


def _dotd(a, b):
    return jnp.dot(a, b, preferred_element_type=F32)


def _dotd_nt(a, b):
    return jax.lax.dot_general(a, b, (((1,), (1,)), ((), ())),
                               preferred_element_type=F32)


def _proj_body(h_ref, wqc_ref, wqu_ref, wk_ref, wv_ref, kn_ref, cs_ref, sn_ref,
               avg_ref, et_ref, q_ref, ke_ref, v_ref, kc_ref, vc_ref):
    hx = h_ref[...]
    cq = _dotd(hx, wqc_ref[...])
    qf = _dotd(cq, wqu_ref[...])
    kx = _dotd(hx, wk_ref[...])
    vx = _dotd(hx, wv_ref[...])
    kx = _rope_rms(kx, cs_ref[...], sn_ref[...], kn_ref[...])
    for hh in range(H):
        q_ref[hh] = qf[:, D * hh:D * (hh + 1)]
    ke_ref[:, :D] = kx
    ke_ref[:, D:] = et_ref[...]
    v_ref[...] = vx
    avg = avg_ref[...]
    kc_ref[...] = _dot(avg, kx)
    vc_ref[...] = _dot(avg, vx)


def _attn_body(q_ref, ke_ref, v_ref, kc_ref, vc_ref, wg_ref, bgp_ref,
               qn_ref, cs_ref, sn_ref, sink_ref, out_ref):
    hh = pl.program_id(0)
    i = pl.program_id(1)
    M = HB * TQ
    cs2 = jnp.concatenate([cs_ref[...]] * HB, axis=0)
    sn2 = jnp.concatenate([sn_ref[...]] * HB, axis=0)
    q_raw = jnp.concatenate([q_ref[j] for j in range(HB)], axis=0)
    q = _rope_rms(q_raw, cs2, sn2, qn_ref[...])
    rowid = _iota((M, 1), 0)
    qpos = i * TQ + jnp.bitwise_and(rowid, TQ - 1)
    sk = sink_ref[HB * hh]
    for j in range(1, HB):
        sk = jnp.where(rowid < j * TQ, sk, sink_ref[HB * hh + j])

    kc = kc_ref[...]
    sblk = _dotd_nt(q, kc) * SCALE
    be = _iota((M, NB), 1) * CR + (CR - 1)
    valid = be <= qpos
    sblk_m = jnp.where(valid, sblk, NEG)
    anyb = qpos >= (CR - 1)

    s_c = jnp.where(anyb, sblk_m, sblk)
    m_c = jnp.maximum(jnp.max(s_c, axis=1, keepdims=True), sk)
    e_c = jnp.exp(s_c - m_c)
    den_c = jnp.sum(e_c, axis=1, keepdims=True) + jnp.exp(sk - m_c)
    comp = _dotd(e_c, vc_ref[...]) / den_c
    comp = jnp.where(anyb, comp, 0.0)

    colid = _iota((M, NB), 1)
    low = (qpos + 1) // CR < SELK
    s_work = sblk_m
    selb = jnp.zeros((M, NB), jnp.bool_)
    for _ in range(SELK):
        m = jnp.max(s_work, axis=1, keepdims=True)
        eqm = jnp.logical_and(s_work == m, m > -1e29)
        selb = jnp.logical_or(selb, eqm)
        s_work = jnp.where(eqm, -jnp.inf, s_work)
    selb = jnp.logical_or(jnp.logical_and(low, colid < SELK),
                          jnp.logical_and(jnp.logical_not(low), selb))
    selb_f = selb.astype(F32)

    av = jnp.concatenate([q * SCALE, (selb_f - 1.0) * 1e30], axis=1)

    def chunk(c, carry, diag):
        m, l, acc = carry
        keb = ke_ref[pl.ds(c * TQ, TQ), :]
        s = _dotd_nt(av, keb)
        if diag:
            jpos = c * TQ + _iota((M, TQ), 1)
            s = jnp.where(jpos <= qpos, s, NEG)
        mnew = jnp.maximum(m, jnp.max(s, axis=1, keepdims=True))
        p = jnp.exp(s - mnew)
        alpha = jnp.exp(m - mnew)
        l = l * alpha + jnp.sum(p, axis=1, keepdims=True)
        vb = v_ref[pl.ds(c * TQ, TQ), :]
        acc = acc * alpha + _dotd(p, vb)
        return mnew, l, acc

    m0 = jnp.full((M, 1), NEG, F32)
    l0 = jnp.zeros((M, 1), F32)
    a0 = jnp.zeros((M, D), F32)
    carry = jax.lax.fori_loop(0, i, lambda c, cr: chunk(c, cr, False),
                              (m0, l0, a0))
    _, l_s, acc_s = chunk(i, carry, True)
    sel = acc_s / l_s

    prev = jnp.maximum(i - 1, 0)
    kb1 = ke_ref[pl.ds(i * TQ, TQ), :D]
    vb1 = v_ref[pl.ds(i * TQ, TQ), :]
    kb0 = ke_ref[pl.ds(prev * TQ, TQ), :D]
    vb0 = v_ref[pl.ds(prev * TQ, TQ), :]
    s1 = _dotd_nt(q, kb1) * SCALE
    jpos1 = i * TQ + _iota((M, TQ), 1)
    s1 = jnp.where(jpos1 <= qpos, s1, NEG)
    s0 = _dotd_nt(q, kb0) * SCALE
    jpos0 = prev * TQ + _iota((M, TQ), 1)
    ok0 = jnp.logical_and(qpos - jpos0 < WIN, i > 0)
    s0 = jnp.where(ok0, s0, NEG)
    m_w = jnp.maximum(jnp.max(s1, axis=1, keepdims=True),
                      jnp.max(s0, axis=1, keepdims=True))
    m_w = jnp.maximum(m_w, sk)
    e0 = jnp.exp(s0 - m_w)
    e1 = jnp.exp(s1 - m_w)
    den_w = (jnp.sum(e0, axis=1, keepdims=True)
             + jnp.sum(e1, axis=1, keepdims=True) + jnp.exp(sk - m_w))
    sw = (_dotd(e0, vb0) + _dotd(e1, vb1)) / den_w

    g = jax.nn.sigmoid(_dotd(q, wg_ref[...]) + bgp_ref[...])
    gs = [g[:, j:j + 1] for j in range(3)]
    gsum = jnp.maximum(gs[0] + gs[1] + gs[2], 1e-6)
    mixed = (gs[0] * comp + gs[1] * sel + gs[2] * sw) / gsum
    for j in range(HB):
        out_ref[j] = mixed[j * TQ:(j + 1) * TQ]


def _out_body(x_ref, w1_ref, w2_ref, o_ref):
    xs = [x_ref[hh] for hh in range(H)]
    x0 = jnp.concatenate(xs[:H // G], axis=1)
    x1 = jnp.concatenate(xs[H // G:], axis=1)
    y0 = _dotd(x0, w1_ref[0])
    y1 = _dotd(x1, w1_ref[1])
    o_ref[...] = _dotd(y0, w2_ref[:INTER, :]) + _dotd(y1, w2_ref[INTER:, :])


def kernel(h, Wq_comp, Wq_up, Wk, Wv, qn_w, kn_w, Wg, bg, sink, Wo1, Wo2):
    h2 = h[0]
    wqcT = Wq_comp.T
    wquT = Wq_up.T
    wkT = Wk.T
    wvT = Wv.T
    kn2 = kn_w[None, :]
    qn2 = qn_w[None, :]
    wg_pad = jnp.zeros((D, 8), F32).at[:, :3].set(Wg.T)
    bg_pad = jnp.zeros((1, 8), F32).at[0, :3].set(bg)
    et_m = (jnp.arange(T)[:, None] // CR == jnp.arange(NB)[None, :]).astype(F32)
    pos = jnp.arange(T, dtype=F32)
    inv = 1.0 / (THETA ** (jnp.arange(0, ROPE, 2, dtype=F32) / ROPE))
    fr = pos[:, None] * inv[None, :]
    cs_t, sn_t = jnp.cos(fr), jnp.sin(fr)
    avg_m = ((jnp.arange(TQ)[None, :] // CR == jnp.arange(TQ // CR)[:, None])
             .astype(F32) / CR)

    q, ke, v, kc, vc = pl.pallas_call(
        _proj_body,
        grid=(NT,),
        in_specs=[
            pl.BlockSpec((TQ, HID), lambda i: (i, 0)),
            pl.BlockSpec((HID, QCD), lambda i: (0, 0)),
            pl.BlockSpec((QCD, HID), lambda i: (0, 0)),
            pl.BlockSpec((HID, D), lambda i: (0, 0)),
            pl.BlockSpec((HID, D), lambda i: (0, 0)),
            pl.BlockSpec((1, D), lambda i: (0, 0)),
            pl.BlockSpec((TQ, HALF), lambda i: (i, 0)),
            pl.BlockSpec((TQ, HALF), lambda i: (i, 0)),
            pl.BlockSpec((TQ // CR, TQ), lambda i: (0, 0)),
            pl.BlockSpec((TQ, NB), lambda i: (i, 0)),
        ],
        out_specs=[
            pl.BlockSpec((H, TQ, D), lambda i: (0, i, 0)),
            pl.BlockSpec((TQ, 2 * D), lambda i: (i, 0)),
            pl.BlockSpec((TQ, D), lambda i: (i, 0)),
            pl.BlockSpec((TQ // CR, D), lambda i: (i, 0)),
            pl.BlockSpec((TQ // CR, D), lambda i: (i, 0)),
        ],
        out_shape=[
            jax.ShapeDtypeStruct((H, T, D), F32),
            jax.ShapeDtypeStruct((T, 2 * D), F32),
            jax.ShapeDtypeStruct((T, D), F32),
            jax.ShapeDtypeStruct((NB, D), F32),
            jax.ShapeDtypeStruct((NB, D), F32),
        ],
    )(h2, wqcT, wquT, wkT, wvT, kn2, cs_t, sn_t, avg_m, et_m)

    mixed = pl.pallas_call(
        _attn_body,
        grid=(H // HB, NT),
        in_specs=[
            pl.BlockSpec((HB, TQ, D), lambda hh, i: (hh, i, 0)),
            pl.BlockSpec((T, 2 * D), lambda hh, i: (0, 0)),
            pl.BlockSpec((T, D), lambda hh, i: (0, 0)),
            pl.BlockSpec((NB, D), lambda hh, i: (0, 0)),
            pl.BlockSpec((NB, D), lambda hh, i: (0, 0)),
            pl.BlockSpec((D, 8), lambda hh, i: (0, 0)),
            pl.BlockSpec((1, 8), lambda hh, i: (0, 0)),
            pl.BlockSpec((1, D), lambda hh, i: (0, 0)),
            pl.BlockSpec((TQ, HALF), lambda hh, i: (i, 0)),
            pl.BlockSpec((TQ, HALF), lambda hh, i: (i, 0)),
            pl.BlockSpec(memory_space=pltpu.SMEM),
        ],
        out_specs=pl.BlockSpec((HB, TQ, D), lambda hh, i: (hh, i, 0)),
        out_shape=jax.ShapeDtypeStruct((H, T, D), F32),
    )(q, ke, v, kc, vc, wg_pad, bg_pad, qn2, cs_t, sn_t, sink)

    w1t = jnp.transpose(Wo1, (0, 2, 1))
    w2t = Wo2.T
    y = pl.pallas_call(
        _out_body,
        grid=(NT,),
        in_specs=[
            pl.BlockSpec((H, TQ, D), lambda i: (0, i, 0)),
            pl.BlockSpec((G, QCD, INTER), lambda i: (0, 0, 0)),
            pl.BlockSpec((G * INTER, HID), lambda i: (0, 0)),
        ],
        out_specs=pl.BlockSpec((TQ, HID), lambda i: (i, 0)),
        out_shape=jax.ShapeDtypeStruct((T, HID), F32),
    )(mixed, w1t, w2t)
    return y[None]


# speedup vs baseline: 2.5080x; 1.1673x over previous
import math

import jax
import jax.numpy as jnp
from jax.experimental import pallas as pl
from jax.experimental.pallas import tpu as pltpu

T = 2048
HID = 768
H = 12
D = 64
QCD = 384
ROPE = 32
HALF = ROPE // 2
THETA = 10000.0
WIN = 256
CR = 32
SELK = 16
G = 2
INTER = 1024

TQ = 256
NT = T // TQ
NB = T // CR
HB = 4
SCALE = 1.0 / math.sqrt(D)
NEG = -1e30
F32 = jnp.float32


def _iota(shape, dim):
    return jax.lax.broadcasted_iota(jnp.int32, shape, dim)


def _rope_rms(x, cos, sin, w):
    x1 = x[:, :HALF]
    x2 = x[:, HALF:ROPE]
    rot = jnp.concatenate([x1 * cos - x2 * sin, x1 * sin + x2 * cos], axis=1)
    x = jnp.concatenate([rot, x[:, ROPE:]], axis=1)
    ms = jnp.mean(x * x, axis=1, keepdims=True)
    return x * jax.lax.rsqrt(ms + 1e-6) * w


def _dot(a, b):
    return jnp.dot(a, b, preferred_element_type=F32,
                   precision=jax.lax.Precision.HIGHEST)


def _dotd(a, b):
    return jnp.dot(a, b, preferred_element_type=F32)


def _dotd_nt(a, b):
    return jax.lax.dot_general(a, b, (((1,), (1,)), ((), ())),
                               preferred_element_type=F32)


def _proj_body(h_ref, wqc_ref, wqu_ref, wk_ref, wv_ref, kn_ref, cs_ref, sn_ref,
               avg_ref, et_ref, q_ref, ke_ref, v_ref, kc_ref, vc_ref):
    hx = h_ref[...]
    cq = _dotd(hx, wqc_ref[...])
    qf = _dotd(cq, wqu_ref[...])
    kx = _dotd(hx, wk_ref[...])
    vx = _dotd(hx, wv_ref[...])
    kx = _rope_rms(kx, cs_ref[...], sn_ref[...], kn_ref[...])
    for hh in range(H):
        q_ref[hh] = qf[:, D * hh:D * (hh + 1)]
    ke_ref[:, :D] = kx
    ke_ref[:, D:] = et_ref[...]
    v_ref[...] = vx
    avg = avg_ref[...]
    kc_ref[...] = _dot(avg, kx)
    vc_ref[...] = _dot(avg, vx)


def _attn_body(q_ref, ke_ref, v_ref, kc_ref, vc_ref, wg_ref, bgp_ref,
               qn_ref, cs_ref, sn_ref, sink_ref, out_ref):
    hh = pl.program_id(0)
    i = pl.program_id(1)
    M = HB * TQ
    cs2 = jnp.concatenate([cs_ref[...]] * HB, axis=0)
    sn2 = jnp.concatenate([sn_ref[...]] * HB, axis=0)
    q_raw = jnp.concatenate([q_ref[j] for j in range(HB)], axis=0)
    q = _rope_rms(q_raw, cs2, sn2, qn_ref[...])
    rowid = _iota((M, 1), 0)
    qpos = i * TQ + jnp.bitwise_and(rowid, TQ - 1)
    sk = sink_ref[HB * hh]
    for j in range(1, HB):
        sk = jnp.where(rowid < j * TQ, sk, sink_ref[HB * hh + j])

    kc = kc_ref[...]
    sblk = _dotd_nt(q, kc) * SCALE
    be = _iota((M, NB), 1) * CR + (CR - 1)
    valid = be <= qpos
    sblk_m = jnp.where(valid, sblk, NEG)
    anyb = qpos >= (CR - 1)

    s_c = jnp.where(anyb, sblk_m, sblk)
    m_c = jnp.maximum(jnp.max(s_c, axis=1, keepdims=True), sk)
    e_c = jnp.exp(s_c - m_c)
    den_c = jnp.sum(e_c, axis=1, keepdims=True) + jnp.exp(sk - m_c)
    comp = _dotd(e_c, vc_ref[...]) / den_c
    comp = jnp.where(anyb, comp, 0.0)

    colid = _iota((M, NB), 1)
    low = (qpos + 1) // CR < SELK
    s_work = sblk_m
    selb = jnp.zeros((M, NB), jnp.bool_)
    for _ in range(SELK):
        m = jnp.max(s_work, axis=1, keepdims=True)
        eqm = jnp.logical_and(s_work == m, m > -1e29)
        selb = jnp.logical_or(selb, eqm)
        s_work = jnp.where(eqm, -jnp.inf, s_work)
    selb = jnp.logical_or(jnp.logical_and(low, colid < SELK),
                          jnp.logical_and(jnp.logical_not(low), selb))
    selb_f = selb.astype(F32)

    av = jnp.concatenate([q * SCALE, (selb_f - 1.0) * 1e30], axis=1)

    def chunk(c, carry, diag):
        m, l, acc = carry
        keb = ke_ref[pl.ds(c * TQ, TQ), :]
        s = _dotd_nt(av, keb)
        if diag:
            jpos = c * TQ + _iota((M, TQ), 1)
            s = jnp.where(jpos <= qpos, s, NEG)
        mnew = jnp.maximum(m, jnp.max(s, axis=1, keepdims=True))
        p = jnp.exp(s - mnew)
        alpha = jnp.exp(m - mnew)
        l = l * alpha + jnp.sum(p, axis=1, keepdims=True)
        vb = v_ref[pl.ds(c * TQ, TQ), :]
        acc = acc * alpha + _dotd(p, vb)
        return mnew, l, acc

    m0 = jnp.full((M, 1), NEG, F32)
    l0 = jnp.zeros((M, 1), F32)
    a0 = jnp.zeros((M, D), F32)
    carry = jax.lax.fori_loop(0, i, lambda c, cr: chunk(c, cr, False),
                              (m0, l0, a0))
    _, l_s, acc_s = chunk(i, carry, True)
    sel = acc_s / l_s

    prev = jnp.maximum(i - 1, 0)
    kb1 = ke_ref[pl.ds(i * TQ, TQ), :D]
    vb1 = v_ref[pl.ds(i * TQ, TQ), :]
    kb0 = ke_ref[pl.ds(prev * TQ, TQ), :D]
    vb0 = v_ref[pl.ds(prev * TQ, TQ), :]
    s1 = _dotd_nt(q, kb1) * SCALE
    jpos1 = i * TQ + _iota((M, TQ), 1)
    s1 = jnp.where(jpos1 <= qpos, s1, NEG)
    s0 = _dotd_nt(q, kb0) * SCALE
    jpos0 = prev * TQ + _iota((M, TQ), 1)
    ok0 = jnp.logical_and(qpos - jpos0 < WIN, i > 0)
    s0 = jnp.where(ok0, s0, NEG)
    m_w = jnp.maximum(jnp.max(s1, axis=1, keepdims=True),
                      jnp.max(s0, axis=1, keepdims=True))
    m_w = jnp.maximum(m_w, sk)
    e0 = jnp.exp(s0 - m_w)
    e1 = jnp.exp(s1 - m_w)
    den_w = (jnp.sum(e0, axis=1, keepdims=True)
             + jnp.sum(e1, axis=1, keepdims=True) + jnp.exp(sk - m_w))
    sw = (_dotd(e0, vb0) + _dotd(e1, vb1)) / den_w

    g = jax.nn.sigmoid(_dotd(q, wg_ref[...]) + bgp_ref[...])
    gs = [g[:, j:j + 1] for j in range(3)]
    gsum = jnp.maximum(gs[0] + gs[1] + gs[2], 1e-6)
    mixed = (gs[0] * comp + gs[1] * sel + gs[2] * sw) / gsum
    for j in range(HB):
        out_ref[j] = mixed[j * TQ:(j + 1) * TQ]


def _out_body(x_ref, w1_ref, w2_ref, o_ref):
    xs = [x_ref[hh] for hh in range(H)]
    x0 = jnp.concatenate(xs[:H // G], axis=1)
    x1 = jnp.concatenate(xs[H // G:], axis=1)
    y0 = _dotd(x0, w1_ref[0])
    y1 = _dotd(x1, w1_ref[1])
    o_ref[...] = _dotd(y0, w2_ref[:INTER, :]) + _dotd(y1, w2_ref[INTER:, :])


def kernel(h, Wq_comp, Wq_up, Wk, Wv, qn_w, kn_w, Wg, bg, sink, Wo1, Wo2):
    h2 = h[0]
    wqcT = Wq_comp.T
    wquT = Wq_up.T
    wkT = Wk.T
    wvT = Wv.T
    kn2 = kn_w[None, :]
    qn2 = qn_w[None, :]
    wg_pad = jnp.zeros((D, 8), F32).at[:, :3].set(Wg.T)
    bg_pad = jnp.zeros((1, 8), F32).at[0, :3].set(bg)
    et_m = (jnp.arange(T)[:, None] // CR == jnp.arange(NB)[None, :]).astype(F32)
    pos = jnp.arange(T, dtype=F32)
    inv = 1.0 / (THETA ** (jnp.arange(0, ROPE, 2, dtype=F32) / ROPE))
    fr = pos[:, None] * inv[None, :]
    cs_t, sn_t = jnp.cos(fr), jnp.sin(fr)
    avg_m = ((jnp.arange(TQ)[None, :] // CR == jnp.arange(TQ // CR)[:, None])
             .astype(F32) / CR)

    q, ke, v, kc, vc = pl.pallas_call(
        _proj_body,
        grid=(NT,),
        in_specs=[
            pl.BlockSpec((TQ, HID), lambda i: (i, 0)),
            pl.BlockSpec((HID, QCD), lambda i: (0, 0)),
            pl.BlockSpec((QCD, HID), lambda i: (0, 0)),
            pl.BlockSpec((HID, D), lambda i: (0, 0)),
            pl.BlockSpec((HID, D), lambda i: (0, 0)),
            pl.BlockSpec((1, D), lambda i: (0, 0)),
            pl.BlockSpec((TQ, HALF), lambda i: (i, 0)),
            pl.BlockSpec((TQ, HALF), lambda i: (i, 0)),
            pl.BlockSpec((TQ // CR, TQ), lambda i: (0, 0)),
            pl.BlockSpec((TQ, NB), lambda i: (i, 0)),
        ],
        out_specs=[
            pl.BlockSpec((H, TQ, D), lambda i: (0, i, 0)),
            pl.BlockSpec((TQ, 2 * D), lambda i: (i, 0)),
            pl.BlockSpec((TQ, D), lambda i: (i, 0)),
            pl.BlockSpec((TQ // CR, D), lambda i: (i, 0)),
            pl.BlockSpec((TQ // CR, D), lambda i: (i, 0)),
        ],
        out_shape=[
            jax.ShapeDtypeStruct((H, T, D), F32),
            jax.ShapeDtypeStruct((T, 2 * D), F32),
            jax.ShapeDtypeStruct((T, D), F32),
            jax.ShapeDtypeStruct((NB, D), F32),
            jax.ShapeDtypeStruct((NB, D), F32),
        ],
    )(h2, wqcT, wquT, wkT, wvT, kn2, cs_t, sn_t, avg_m, et_m)

    mixed = pl.pallas_call(
        _attn_body,
        grid=(H // HB, NT),
        in_specs=[
            pl.BlockSpec((HB, TQ, D), lambda hh, i: (hh, i, 0)),
            pl.BlockSpec((T, 2 * D), lambda hh, i: (0, 0)),
            pl.BlockSpec((T, D), lambda hh, i: (0, 0)),
            pl.BlockSpec((NB, D), lambda hh, i: (0, 0)),
            pl.BlockSpec((NB, D), lambda hh, i: (0, 0)),
            pl.BlockSpec((D, 8), lambda hh, i: (0, 0)),
            pl.BlockSpec((1, 8), lambda hh, i: (0, 0)),
            pl.BlockSpec((1, D), lambda hh, i: (0, 0)),
            pl.BlockSpec((TQ, HALF), lambda hh, i: (i, 0)),
            pl.BlockSpec((TQ, HALF), lambda hh, i: (i, 0)),
            pl.BlockSpec(memory_space=pltpu.SMEM),
        ],
        out_specs=pl.BlockSpec((HB, TQ, D), lambda hh, i: (hh, i, 0)),
        out_shape=jax.ShapeDtypeStruct((H, T, D), F32),
    )(q, ke, v, kc, vc, wg_pad, bg_pad, qn2, cs_t, sn_t, sink)

    w1t = jnp.transpose(Wo1, (0, 2, 1))
    w2t = Wo2.T
    y = pl.pallas_call(
        _out_body,
        grid=(NT,),
        in_specs=[
            pl.BlockSpec((H, TQ, D), lambda i: (0, i, 0)),
            pl.BlockSpec((G, QCD, INTER), lambda i: (0, 0, 0)),
            pl.BlockSpec((G * INTER, HID), lambda i: (0, 0)),
        ],
        out_specs=pl.BlockSpec((TQ, HID), lambda i: (i, 0)),
        out_shape=jax.ShapeDtypeStruct((T, HID), F32),
    )(mixed, w1t, w2t)
    return y[None]


# speedup vs baseline: 2.6565x; 1.0592x over previous
import math

import jax
import jax.numpy as jnp
from jax.experimental import pallas as pl
from jax.experimental.pallas import tpu as pltpu

T = 2048
HID = 768
H = 12
D = 64
QCD = 384
ROPE = 32
HALF = ROPE // 2
THETA = 10000.0
WIN = 256
CR = 32
SELK = 16
G = 2
INTER = 1024

TQ = 256
NT = T // TQ
NB = T // CR
HB = 6
SCALE = 1.0 / math.sqrt(D)
NEG = -1e30
F32 = jnp.float32


def _iota(shape, dim):
    return jax.lax.broadcasted_iota(jnp.int32, shape, dim)


def _rope_rms(x, cos, sin, w):
    x1 = x[:, :HALF]
    x2 = x[:, HALF:ROPE]
    rot = jnp.concatenate([x1 * cos - x2 * sin, x1 * sin + x2 * cos], axis=1)
    x = jnp.concatenate([rot, x[:, ROPE:]], axis=1)
    ms = jnp.mean(x * x, axis=1, keepdims=True)
    return x * jax.lax.rsqrt(ms + 1e-6) * w


def _dot(a, b):
    return jnp.dot(a, b, preferred_element_type=F32,
                   precision=jax.lax.Precision.HIGHEST)


def _dotd(a, b):
    return jnp.dot(a, b, preferred_element_type=F32)


def _dotd_nt(a, b):
    return jax.lax.dot_general(a, b, (((1,), (1,)), ((), ())),
                               preferred_element_type=F32)


def _proj_body(h_ref, wqc_ref, wqu_ref, wk_ref, wv_ref, kn_ref, cs_ref, sn_ref,
               avg_ref, et_ref, q_ref, ke_ref, v_ref, kc_ref, vc_ref):
    hx = h_ref[...]
    cq = _dotd(hx, wqc_ref[...])
    qf = _dotd(cq, wqu_ref[...])
    kx = _dotd(hx, wk_ref[...])
    vx = _dotd(hx, wv_ref[...])
    kx = _rope_rms(kx, cs_ref[...], sn_ref[...], kn_ref[...])
    for hh in range(H):
        q_ref[hh] = qf[:, D * hh:D * (hh + 1)]
    ke_ref[:, :D] = kx
    ke_ref[:, D:] = et_ref[...]
    v_ref[...] = vx
    avg = avg_ref[...]
    kc_ref[...] = _dot(avg, kx)
    vc_ref[...] = _dot(avg, vx)


def _attn_body(q_ref, ke_ref, v_ref, kc_ref, vc_ref, wg_ref, bgp_ref,
               qn_ref, cs_ref, sn_ref, sink_ref, out_ref):
    hh = pl.program_id(0)
    i = pl.program_id(1)
    M = HB * TQ
    cs2 = jnp.concatenate([cs_ref[...]] * HB, axis=0)
    sn2 = jnp.concatenate([sn_ref[...]] * HB, axis=0)
    q_raw = jnp.concatenate([q_ref[j] for j in range(HB)], axis=0)
    q = _rope_rms(q_raw, cs2, sn2, qn_ref[...])
    rowid = _iota((M, 1), 0)
    qpos = i * TQ + jnp.bitwise_and(rowid, TQ - 1)
    sk = sink_ref[HB * hh]
    for j in range(1, HB):
        sk = jnp.where(rowid < j * TQ, sk, sink_ref[HB * hh + j])

    kc = kc_ref[...]
    sblk = _dotd_nt(q, kc) * SCALE
    be = _iota((M, NB), 1) * CR + (CR - 1)
    valid = be <= qpos
    sblk_m = jnp.where(valid, sblk, NEG)
    anyb = qpos >= (CR - 1)

    s_c = jnp.where(anyb, sblk_m, sblk)
    m_c = jnp.maximum(jnp.max(s_c, axis=1, keepdims=True), sk)
    e_c = jnp.exp(s_c - m_c)
    den_c = jnp.sum(e_c, axis=1, keepdims=True) + jnp.exp(sk - m_c)
    comp = _dotd(e_c, vc_ref[...]) / den_c
    comp = jnp.where(anyb, comp, 0.0)

    colid = _iota((M, NB), 1)
    low = (qpos + 1) // CR < SELK
    s_work = sblk_m
    selb = jnp.zeros((M, NB), jnp.bool_)
    for _ in range(SELK):
        m = jnp.max(s_work, axis=1, keepdims=True)
        eqm = jnp.logical_and(s_work == m, m > -1e29)
        selb = jnp.logical_or(selb, eqm)
        s_work = jnp.where(eqm, -jnp.inf, s_work)
    selb = jnp.logical_or(jnp.logical_and(low, colid < SELK),
                          jnp.logical_and(jnp.logical_not(low), selb))
    selb_f = selb.astype(F32)

    av = jnp.concatenate([q * SCALE, (selb_f - 1.0) * 1e30], axis=1)

    def chunk(c, carry, diag):
        m, l, acc = carry
        keb = ke_ref[pl.ds(c * TQ, TQ), :]
        s = _dotd_nt(av, keb)
        if diag:
            jpos = c * TQ + _iota((M, TQ), 1)
            s = jnp.where(jpos <= qpos, s, NEG)
        mnew = jnp.maximum(m, jnp.max(s, axis=1, keepdims=True))
        p = jnp.exp(s - mnew)
        alpha = jnp.exp(m - mnew)
        l = l * alpha + jnp.sum(p, axis=1, keepdims=True)
        vb = v_ref[pl.ds(c * TQ, TQ), :]
        acc = acc * alpha + _dotd(p, vb)
        return mnew, l, acc

    m0 = jnp.full((M, 1), NEG, F32)
    l0 = jnp.zeros((M, 1), F32)
    a0 = jnp.zeros((M, D), F32)
    carry = jax.lax.fori_loop(0, i, lambda c, cr: chunk(c, cr, False),
                              (m0, l0, a0))
    _, l_s, acc_s = chunk(i, carry, True)
    sel = acc_s / l_s

    prev = jnp.maximum(i - 1, 0)
    kb1 = ke_ref[pl.ds(i * TQ, TQ), :D]
    vb1 = v_ref[pl.ds(i * TQ, TQ), :]
    kb0 = ke_ref[pl.ds(prev * TQ, TQ), :D]
    vb0 = v_ref[pl.ds(prev * TQ, TQ), :]
    s1 = _dotd_nt(q, kb1) * SCALE
    jpos1 = i * TQ + _iota((M, TQ), 1)
    s1 = jnp.where(jpos1 <= qpos, s1, NEG)
    s0 = _dotd_nt(q, kb0) * SCALE
    jpos0 = prev * TQ + _iota((M, TQ), 1)
    ok0 = jnp.logical_and(qpos - jpos0 < WIN, i > 0)
    s0 = jnp.where(ok0, s0, NEG)
    m_w = jnp.maximum(jnp.max(s1, axis=1, keepdims=True),
                      jnp.max(s0, axis=1, keepdims=True))
    m_w = jnp.maximum(m_w, sk)
    e0 = jnp.exp(s0 - m_w)
    e1 = jnp.exp(s1 - m_w)
    den_w = (jnp.sum(e0, axis=1, keepdims=True)
             + jnp.sum(e1, axis=1, keepdims=True) + jnp.exp(sk - m_w))
    sw = (_dotd(e0, vb0) + _dotd(e1, vb1)) / den_w

    g = jax.nn.sigmoid(_dotd(q, wg_ref[...]) + bgp_ref[...])
    gs = [g[:, j:j + 1] for j in range(3)]
    gsum = jnp.maximum(gs[0] + gs[1] + gs[2], 1e-6)
    mixed = (gs[0] * comp + gs[1] * sel + gs[2] * sw) / gsum
    for j in range(HB):
        out_ref[j] = mixed[j * TQ:(j + 1) * TQ]


def _out_body(x_ref, w1_ref, w2_ref, o_ref):
    xs = [x_ref[hh] for hh in range(H)]
    x0 = jnp.concatenate(xs[:H // G], axis=1)
    x1 = jnp.concatenate(xs[H // G:], axis=1)
    y0 = _dotd(x0, w1_ref[0])
    y1 = _dotd(x1, w1_ref[1])
    o_ref[...] = _dotd(y0, w2_ref[:INTER, :]) + _dotd(y1, w2_ref[INTER:, :])


def kernel(h, Wq_comp, Wq_up, Wk, Wv, qn_w, kn_w, Wg, bg, sink, Wo1, Wo2):
    h2 = h[0]
    wqcT = Wq_comp.T
    wquT = Wq_up.T
    wkT = Wk.T
    wvT = Wv.T
    kn2 = kn_w[None, :]
    qn2 = qn_w[None, :]
    wg_pad = jnp.zeros((D, 8), F32).at[:, :3].set(Wg.T)
    bg_pad = jnp.zeros((1, 8), F32).at[0, :3].set(bg)
    et_m = (jnp.arange(T)[:, None] // CR == jnp.arange(NB)[None, :]).astype(F32)
    pos = jnp.arange(T, dtype=F32)
    inv = 1.0 / (THETA ** (jnp.arange(0, ROPE, 2, dtype=F32) / ROPE))
    fr = pos[:, None] * inv[None, :]
    cs_t, sn_t = jnp.cos(fr), jnp.sin(fr)
    avg_m = ((jnp.arange(TQ)[None, :] // CR == jnp.arange(TQ // CR)[:, None])
             .astype(F32) / CR)

    q, ke, v, kc, vc = pl.pallas_call(
        _proj_body,
        grid=(NT,),
        in_specs=[
            pl.BlockSpec((TQ, HID), lambda i: (i, 0)),
            pl.BlockSpec((HID, QCD), lambda i: (0, 0)),
            pl.BlockSpec((QCD, HID), lambda i: (0, 0)),
            pl.BlockSpec((HID, D), lambda i: (0, 0)),
            pl.BlockSpec((HID, D), lambda i: (0, 0)),
            pl.BlockSpec((1, D), lambda i: (0, 0)),
            pl.BlockSpec((TQ, HALF), lambda i: (i, 0)),
            pl.BlockSpec((TQ, HALF), lambda i: (i, 0)),
            pl.BlockSpec((TQ // CR, TQ), lambda i: (0, 0)),
            pl.BlockSpec((TQ, NB), lambda i: (i, 0)),
        ],
        out_specs=[
            pl.BlockSpec((H, TQ, D), lambda i: (0, i, 0)),
            pl.BlockSpec((TQ, 2 * D), lambda i: (i, 0)),
            pl.BlockSpec((TQ, D), lambda i: (i, 0)),
            pl.BlockSpec((TQ // CR, D), lambda i: (i, 0)),
            pl.BlockSpec((TQ // CR, D), lambda i: (i, 0)),
        ],
        out_shape=[
            jax.ShapeDtypeStruct((H, T, D), F32),
            jax.ShapeDtypeStruct((T, 2 * D), F32),
            jax.ShapeDtypeStruct((T, D), F32),
            jax.ShapeDtypeStruct((NB, D), F32),
            jax.ShapeDtypeStruct((NB, D), F32),
        ],
    )(h2, wqcT, wquT, wkT, wvT, kn2, cs_t, sn_t, avg_m, et_m)

    mixed = pl.pallas_call(
        _attn_body,
        grid=(H // HB, NT),
        in_specs=[
            pl.BlockSpec((HB, TQ, D), lambda hh, i: (hh, i, 0)),
            pl.BlockSpec((T, 2 * D), lambda hh, i: (0, 0)),
            pl.BlockSpec((T, D), lambda hh, i: (0, 0)),
            pl.BlockSpec((NB, D), lambda hh, i: (0, 0)),
            pl.BlockSpec((NB, D), lambda hh, i: (0, 0)),
            pl.BlockSpec((D, 8), lambda hh, i: (0, 0)),
            pl.BlockSpec((1, 8), lambda hh, i: (0, 0)),
            pl.BlockSpec((1, D), lambda hh, i: (0, 0)),
            pl.BlockSpec((TQ, HALF), lambda hh, i: (i, 0)),
            pl.BlockSpec((TQ, HALF), lambda hh, i: (i, 0)),
            pl.BlockSpec(memory_space=pltpu.SMEM),
        ],
        out_specs=pl.BlockSpec((HB, TQ, D), lambda hh, i: (hh, i, 0)),
        out_shape=jax.ShapeDtypeStruct((H, T, D), F32),
    )(q, ke, v, kc, vc, wg_pad, bg_pad, qn2, cs_t, sn_t, sink)

    w1t = jnp.transpose(Wo1, (0, 2, 1))
    w2t = Wo2.T
    y = pl.pallas_call(
        _out_body,
        grid=(NT,),
        in_specs=[
            pl.BlockSpec((H, TQ, D), lambda i: (0, i, 0)),
            pl.BlockSpec((G, QCD, INTER), lambda i: (0, 0, 0)),
            pl.BlockSpec((G * INTER, HID), lambda i: (0, 0)),
        ],
        out_specs=pl.BlockSpec((TQ, HID), lambda i: (i, 0)),
        out_shape=jax.ShapeDtypeStruct((T, HID), F32),
    )(mixed, w1t, w2t)
    return y[None]


# speedup vs baseline: 2.7165x; 1.0226x over previous
import math

import jax
import jax.numpy as jnp
from jax.experimental import pallas as pl
from jax.experimental.pallas import tpu as pltpu

T = 2048
HID = 768
H = 12
D = 64
QCD = 384
ROPE = 32
HALF = ROPE // 2
THETA = 10000.0
WIN = 256
CR = 32
SELK = 16
G = 2
INTER = 1024

TQ = 256
NT = T // TQ
NB = T // CR
HB = 12
SCALE = 1.0 / math.sqrt(D)
NEG = -1e30
F32 = jnp.float32


def _iota(shape, dim):
    return jax.lax.broadcasted_iota(jnp.int32, shape, dim)


def _rope_rms(x, cos, sin, w):
    x1 = x[:, :HALF]
    x2 = x[:, HALF:ROPE]
    rot = jnp.concatenate([x1 * cos - x2 * sin, x1 * sin + x2 * cos], axis=1)
    x = jnp.concatenate([rot, x[:, ROPE:]], axis=1)
    ms = jnp.mean(x * x, axis=1, keepdims=True)
    return x * jax.lax.rsqrt(ms + 1e-6) * w


def _dot(a, b):
    return jnp.dot(a, b, preferred_element_type=F32,
                   precision=jax.lax.Precision.HIGHEST)


def _dotd(a, b):
    return jnp.dot(a, b, preferred_element_type=F32)


def _dotd_nt(a, b):
    return jax.lax.dot_general(a, b, (((1,), (1,)), ((), ())),
                               preferred_element_type=F32)


def _proj_body(h_ref, wqc_ref, wqu_ref, wk_ref, wv_ref, kn_ref, cs_ref, sn_ref,
               avg_ref, et_ref, q_ref, ke_ref, v_ref, kc_ref, vc_ref):
    hx = h_ref[...]
    cq = _dotd(hx, wqc_ref[...])
    qf = _dotd(cq, wqu_ref[...])
    kx = _dotd(hx, wk_ref[...])
    vx = _dotd(hx, wv_ref[...])
    kx = _rope_rms(kx, cs_ref[...], sn_ref[...], kn_ref[...])
    for hh in range(H):
        q_ref[hh] = qf[:, D * hh:D * (hh + 1)]
    ke_ref[:, :D] = kx
    ke_ref[:, D:] = et_ref[...]
    v_ref[...] = vx
    avg = avg_ref[...]
    kc_ref[...] = _dot(avg, kx)
    vc_ref[...] = _dot(avg, vx)


def _attn_body(q_ref, ke_ref, v_ref, kc_ref, vc_ref, wg_ref, bgp_ref,
               qn_ref, cs_ref, sn_ref, sink_ref, out_ref):
    hh = pl.program_id(0)
    i = pl.program_id(1)
    M = HB * TQ
    cs2 = jnp.concatenate([cs_ref[...]] * HB, axis=0)
    sn2 = jnp.concatenate([sn_ref[...]] * HB, axis=0)
    q_raw = jnp.concatenate([q_ref[j] for j in range(HB)], axis=0)
    q = _rope_rms(q_raw, cs2, sn2, qn_ref[...])
    rowid = _iota((M, 1), 0)
    qpos = i * TQ + jnp.bitwise_and(rowid, TQ - 1)
    sk = sink_ref[HB * hh]
    for j in range(1, HB):
        sk = jnp.where(rowid < j * TQ, sk, sink_ref[HB * hh + j])

    kc = kc_ref[...]
    sblk = _dotd_nt(q, kc) * SCALE
    be = _iota((M, NB), 1) * CR + (CR - 1)
    valid = be <= qpos
    sblk_m = jnp.where(valid, sblk, NEG)
    anyb = qpos >= (CR - 1)

    s_c = jnp.where(anyb, sblk_m, sblk)
    m_c = jnp.maximum(jnp.max(s_c, axis=1, keepdims=True), sk)
    e_c = jnp.exp(s_c - m_c)
    den_c = jnp.sum(e_c, axis=1, keepdims=True) + jnp.exp(sk - m_c)
    comp = _dotd(e_c, vc_ref[...]) / den_c
    comp = jnp.where(anyb, comp, 0.0)

    colid = _iota((M, NB), 1)
    low = (qpos + 1) // CR < SELK
    s_work = sblk_m
    selb = jnp.zeros((M, NB), jnp.bool_)
    for _ in range(SELK):
        m = jnp.max(s_work, axis=1, keepdims=True)
        eqm = jnp.logical_and(s_work == m, m > -1e29)
        selb = jnp.logical_or(selb, eqm)
        s_work = jnp.where(eqm, -jnp.inf, s_work)
    selb = jnp.logical_or(jnp.logical_and(low, colid < SELK),
                          jnp.logical_and(jnp.logical_not(low), selb))
    selb_f = selb.astype(F32)

    av = jnp.concatenate([q * SCALE, (selb_f - 1.0) * 1e30], axis=1)

    def chunk(c, carry, diag):
        m, l, acc = carry
        keb = ke_ref[pl.ds(c * TQ, TQ), :]
        s = _dotd_nt(av, keb)
        if diag:
            jpos = c * TQ + _iota((M, TQ), 1)
            s = jnp.where(jpos <= qpos, s, NEG)
        mnew = jnp.maximum(m, jnp.max(s, axis=1, keepdims=True))
        p = jnp.exp(s - mnew)
        alpha = jnp.exp(m - mnew)
        l = l * alpha + jnp.sum(p, axis=1, keepdims=True)
        vb = v_ref[pl.ds(c * TQ, TQ), :]
        acc = acc * alpha + _dotd(p, vb)
        return mnew, l, acc

    m0 = jnp.full((M, 1), NEG, F32)
    l0 = jnp.zeros((M, 1), F32)
    a0 = jnp.zeros((M, D), F32)
    carry = jax.lax.fori_loop(0, i, lambda c, cr: chunk(c, cr, False),
                              (m0, l0, a0))
    _, l_s, acc_s = chunk(i, carry, True)
    sel = acc_s / l_s

    prev = jnp.maximum(i - 1, 0)
    kb1 = ke_ref[pl.ds(i * TQ, TQ), :D]
    vb1 = v_ref[pl.ds(i * TQ, TQ), :]
    kb0 = ke_ref[pl.ds(prev * TQ, TQ), :D]
    vb0 = v_ref[pl.ds(prev * TQ, TQ), :]
    s1 = _dotd_nt(q, kb1) * SCALE
    jpos1 = i * TQ + _iota((M, TQ), 1)
    s1 = jnp.where(jpos1 <= qpos, s1, NEG)
    s0 = _dotd_nt(q, kb0) * SCALE
    jpos0 = prev * TQ + _iota((M, TQ), 1)
    ok0 = jnp.logical_and(qpos - jpos0 < WIN, i > 0)
    s0 = jnp.where(ok0, s0, NEG)
    m_w = jnp.maximum(jnp.max(s1, axis=1, keepdims=True),
                      jnp.max(s0, axis=1, keepdims=True))
    m_w = jnp.maximum(m_w, sk)
    e0 = jnp.exp(s0 - m_w)
    e1 = jnp.exp(s1 - m_w)
    den_w = (jnp.sum(e0, axis=1, keepdims=True)
             + jnp.sum(e1, axis=1, keepdims=True) + jnp.exp(sk - m_w))
    sw = (_dotd(e0, vb0) + _dotd(e1, vb1)) / den_w

    g = jax.nn.sigmoid(_dotd(q, wg_ref[...]) + bgp_ref[...])
    gs = [g[:, j:j + 1] for j in range(3)]
    gsum = jnp.maximum(gs[0] + gs[1] + gs[2], 1e-6)
    mixed = (gs[0] * comp + gs[1] * sel + gs[2] * sw) / gsum
    for j in range(HB):
        out_ref[j] = mixed[j * TQ:(j + 1) * TQ]


def _out_body(x_ref, w1_ref, w2_ref, o_ref):
    xs = [x_ref[hh] for hh in range(H)]
    x0 = jnp.concatenate(xs[:H // G], axis=1)
    x1 = jnp.concatenate(xs[H // G:], axis=1)
    y0 = _dotd(x0, w1_ref[0])
    y1 = _dotd(x1, w1_ref[1])
    o_ref[...] = _dotd(y0, w2_ref[:INTER, :]) + _dotd(y1, w2_ref[INTER:, :])


def kernel(h, Wq_comp, Wq_up, Wk, Wv, qn_w, kn_w, Wg, bg, sink, Wo1, Wo2):
    h2 = h[0]
    wqcT = Wq_comp.T
    wquT = Wq_up.T
    wkT = Wk.T
    wvT = Wv.T
    kn2 = kn_w[None, :]
    qn2 = qn_w[None, :]
    wg_pad = jnp.zeros((D, 8), F32).at[:, :3].set(Wg.T)
    bg_pad = jnp.zeros((1, 8), F32).at[0, :3].set(bg)
    et_m = (jnp.arange(T)[:, None] // CR == jnp.arange(NB)[None, :]).astype(F32)
    pos = jnp.arange(T, dtype=F32)
    inv = 1.0 / (THETA ** (jnp.arange(0, ROPE, 2, dtype=F32) / ROPE))
    fr = pos[:, None] * inv[None, :]
    cs_t, sn_t = jnp.cos(fr), jnp.sin(fr)
    avg_m = ((jnp.arange(TQ)[None, :] // CR == jnp.arange(TQ // CR)[:, None])
             .astype(F32) / CR)

    q, ke, v, kc, vc = pl.pallas_call(
        _proj_body,
        grid=(NT,),
        in_specs=[
            pl.BlockSpec((TQ, HID), lambda i: (i, 0)),
            pl.BlockSpec((HID, QCD), lambda i: (0, 0)),
            pl.BlockSpec((QCD, HID), lambda i: (0, 0)),
            pl.BlockSpec((HID, D), lambda i: (0, 0)),
            pl.BlockSpec((HID, D), lambda i: (0, 0)),
            pl.BlockSpec((1, D), lambda i: (0, 0)),
            pl.BlockSpec((TQ, HALF), lambda i: (i, 0)),
            pl.BlockSpec((TQ, HALF), lambda i: (i, 0)),
            pl.BlockSpec((TQ // CR, TQ), lambda i: (0, 0)),
            pl.BlockSpec((TQ, NB), lambda i: (i, 0)),
        ],
        out_specs=[
            pl.BlockSpec((H, TQ, D), lambda i: (0, i, 0)),
            pl.BlockSpec((TQ, 2 * D), lambda i: (i, 0)),
            pl.BlockSpec((TQ, D), lambda i: (i, 0)),
            pl.BlockSpec((TQ // CR, D), lambda i: (i, 0)),
            pl.BlockSpec((TQ // CR, D), lambda i: (i, 0)),
        ],
        out_shape=[
            jax.ShapeDtypeStruct((H, T, D), F32),
            jax.ShapeDtypeStruct((T, 2 * D), F32),
            jax.ShapeDtypeStruct((T, D), F32),
            jax.ShapeDtypeStruct((NB, D), F32),
            jax.ShapeDtypeStruct((NB, D), F32),
        ],
    )(h2, wqcT, wquT, wkT, wvT, kn2, cs_t, sn_t, avg_m, et_m)

    mixed = pl.pallas_call(
        _attn_body,
        grid=(H // HB, NT),
        in_specs=[
            pl.BlockSpec((HB, TQ, D), lambda hh, i: (hh, i, 0)),
            pl.BlockSpec((T, 2 * D), lambda hh, i: (0, 0)),
            pl.BlockSpec((T, D), lambda hh, i: (0, 0)),
            pl.BlockSpec((NB, D), lambda hh, i: (0, 0)),
            pl.BlockSpec((NB, D), lambda hh, i: (0, 0)),
            pl.BlockSpec((D, 8), lambda hh, i: (0, 0)),
            pl.BlockSpec((1, 8), lambda hh, i: (0, 0)),
            pl.BlockSpec((1, D), lambda hh, i: (0, 0)),
            pl.BlockSpec((TQ, HALF), lambda hh, i: (i, 0)),
            pl.BlockSpec((TQ, HALF), lambda hh, i: (i, 0)),
            pl.BlockSpec(memory_space=pltpu.SMEM),
        ],
        out_specs=pl.BlockSpec((HB, TQ, D), lambda hh, i: (hh, i, 0)),
        out_shape=jax.ShapeDtypeStruct((H, T, D), F32),
    )(q, ke, v, kc, vc, wg_pad, bg_pad, qn2, cs_t, sn_t, sink)

    w1t = jnp.transpose(Wo1, (0, 2, 1))
    w2t = Wo2.T
    y = pl.pallas_call(
        _out_body,
        grid=(NT,),
        in_specs=[
            pl.BlockSpec((H, TQ, D), lambda i: (0, i, 0)),
            pl.BlockSpec((G, QCD, INTER), lambda i: (0, 0, 0)),
            pl.BlockSpec((G * INTER, HID), lambda i: (0, 0)),
        ],
        out_specs=pl.BlockSpec((TQ, HID), lambda i: (i, 0)),
        out_shape=jax.ShapeDtypeStruct((T, HID), F32),
    )(mixed, w1t, w2t)
    return y[None]


# speedup vs baseline: 2.8444x; 1.0471x over previous
import math

import jax
import jax.numpy as jnp
from jax.experimental import pallas as pl
from jax.experimental.pallas import tpu as pltpu

T = 2048
HID = 768
H = 12
D = 64
QCD = 384
ROPE = 32
HALF = ROPE // 2
THETA = 10000.0
WIN = 256
CR = 32
SELK = 16
G = 2
INTER = 1024

TQ = 256
NT = T // TQ
NB = T // CR
HB = 12
SCALE = 1.0 / math.sqrt(D)
NEG = -1e30
F32 = jnp.float32
BF16 = jnp.bfloat16


def _iota(shape, dim):
    return jax.lax.broadcasted_iota(jnp.int32, shape, dim)


def _rope_rms(x, cos, sin, w):
    x1 = x[:, :HALF]
    x2 = x[:, HALF:ROPE]
    rot = jnp.concatenate([x1 * cos - x2 * sin, x1 * sin + x2 * cos], axis=1)
    x = jnp.concatenate([rot, x[:, ROPE:]], axis=1)
    ms = jnp.mean(x * x, axis=1, keepdims=True)
    return x * jax.lax.rsqrt(ms + 1e-6) * w


def _dot(a, b):
    return jnp.dot(a, b, preferred_element_type=F32,
                   precision=jax.lax.Precision.HIGHEST)


def _dotd(a, b):
    return jnp.dot(a, b, preferred_element_type=F32)


def _dotd_nt(a, b):
    return jax.lax.dot_general(a, b, (((1,), (1,)), ((), ())),
                               preferred_element_type=F32)


def _proj_body(h_ref, wqc_ref, wqu_ref, wk_ref, wv_ref, kn_ref, cs_ref, sn_ref,
               avg_ref, et_ref, q_ref, ke_ref, v_ref, kc_ref, vc_ref):
    hx = h_ref[...].astype(BF16)
    cq = _dotd(hx, wqc_ref[...])
    qf = _dotd(cq.astype(BF16), wqu_ref[...])
    kx = _dotd(hx, wk_ref[...])
    vx = _dotd(hx, wv_ref[...])
    kx = _rope_rms(kx, cs_ref[...], sn_ref[...], kn_ref[...])
    for hh in range(H):
        q_ref[hh] = qf[:, D * hh:D * (hh + 1)]
    ke_ref[:, :D] = kx.astype(BF16)
    ke_ref[:, D:] = et_ref[...]
    v_ref[...] = vx.astype(BF16)
    avg = avg_ref[...]
    kc_ref[...] = _dot(avg, kx)
    vc_ref[...] = _dot(avg, vx)


def _attn_body(q_ref, ke_ref, v_ref, kc_ref, vc_ref, wg_ref, bgp_ref,
               qn_ref, cs_ref, sn_ref, sink_ref, out_ref):
    hh = pl.program_id(0)
    i = pl.program_id(1)
    M = HB * TQ
    cs2 = jnp.concatenate([cs_ref[...]] * HB, axis=0)
    sn2 = jnp.concatenate([sn_ref[...]] * HB, axis=0)
    q_raw = jnp.concatenate([q_ref[j] for j in range(HB)], axis=0)
    q = _rope_rms(q_raw, cs2, sn2, qn_ref[...])
    rowid = _iota((M, 1), 0)
    qpos = i * TQ + jnp.bitwise_and(rowid, TQ - 1)
    sk = sink_ref[HB * hh]
    for j in range(1, HB):
        sk = jnp.where(rowid < j * TQ, sk, sink_ref[HB * hh + j])

    kc = kc_ref[...]
    sblk = _dotd_nt(q, kc) * SCALE
    be = _iota((M, NB), 1) * CR + (CR - 1)
    valid = be <= qpos
    sblk_m = jnp.where(valid, sblk, NEG)
    anyb = qpos >= (CR - 1)

    s_c = jnp.where(anyb, sblk_m, sblk)
    m_c = jnp.maximum(jnp.max(s_c, axis=1, keepdims=True), sk)
    e_c = jnp.exp(s_c - m_c)
    den_c = jnp.sum(e_c, axis=1, keepdims=True) + jnp.exp(sk - m_c)
    comp = _dotd(e_c, vc_ref[...]) / den_c
    comp = jnp.where(anyb, comp, 0.0)

    colid = _iota((M, NB), 1)
    low = (qpos + 1) // CR < SELK
    s_work = sblk_m
    selb = jnp.zeros((M, NB), jnp.bool_)
    for _ in range(SELK):
        m = jnp.max(s_work, axis=1, keepdims=True)
        eqm = jnp.logical_and(s_work == m, m > -1e29)
        selb = jnp.logical_or(selb, eqm)
        s_work = jnp.where(eqm, -jnp.inf, s_work)
    selb = jnp.logical_or(jnp.logical_and(low, colid < SELK),
                          jnp.logical_and(jnp.logical_not(low), selb))
    selb_f = selb.astype(F32)

    av = jnp.concatenate([q * SCALE, (selb_f - 1.0) * 1e30],
                         axis=1).astype(BF16)

    def chunk(c, carry, diag):
        m, l, acc = carry
        keb = ke_ref[pl.ds(c * TQ, TQ), :]
        s = _dotd_nt(av, keb)
        if diag:
            jpos = c * TQ + _iota((M, TQ), 1)
            s = jnp.where(jpos <= qpos, s, NEG)
        mnew = jnp.maximum(m, jnp.max(s, axis=1, keepdims=True))
        p = jnp.exp(s - mnew)
        alpha = jnp.exp(m - mnew)
        l = l * alpha + jnp.sum(p, axis=1, keepdims=True)
        vb = v_ref[pl.ds(c * TQ, TQ), :]
        acc = acc * alpha + _dotd(p.astype(BF16), vb)
        return mnew, l, acc

    m0 = jnp.full((M, 1), NEG, F32)
    l0 = jnp.zeros((M, 1), F32)
    a0 = jnp.zeros((M, D), F32)
    carry = jax.lax.fori_loop(0, i, lambda c, cr: chunk(c, cr, False),
                              (m0, l0, a0))
    _, l_s, acc_s = chunk(i, carry, True)
    sel = acc_s / l_s

    prev = jnp.maximum(i - 1, 0)
    kb1 = ke_ref[pl.ds(i * TQ, TQ), :D]
    vb1 = v_ref[pl.ds(i * TQ, TQ), :]
    kb0 = ke_ref[pl.ds(prev * TQ, TQ), :D]
    vb0 = v_ref[pl.ds(prev * TQ, TQ), :]
    q16 = q.astype(BF16)
    s1 = _dotd_nt(q16, kb1) * SCALE
    jpos1 = i * TQ + _iota((M, TQ), 1)
    s1 = jnp.where(jpos1 <= qpos, s1, NEG)
    s0 = _dotd_nt(q16, kb0) * SCALE
    jpos0 = prev * TQ + _iota((M, TQ), 1)
    ok0 = jnp.logical_and(qpos - jpos0 < WIN, i > 0)
    s0 = jnp.where(ok0, s0, NEG)
    m_w = jnp.maximum(jnp.max(s1, axis=1, keepdims=True),
                      jnp.max(s0, axis=1, keepdims=True))
    m_w = jnp.maximum(m_w, sk)
    e0 = jnp.exp(s0 - m_w)
    e1 = jnp.exp(s1 - m_w)
    den_w = (jnp.sum(e0, axis=1, keepdims=True)
             + jnp.sum(e1, axis=1, keepdims=True) + jnp.exp(sk - m_w))
    sw = (_dotd(e0.astype(BF16), vb0) + _dotd(e1.astype(BF16), vb1)) / den_w

    g = jax.nn.sigmoid(_dotd(q, wg_ref[...]) + bgp_ref[...])
    gs = [g[:, j:j + 1] for j in range(3)]
    gsum = jnp.maximum(gs[0] + gs[1] + gs[2], 1e-6)
    mixed = (gs[0] * comp + gs[1] * sel + gs[2] * sw) / gsum
    for j in range(HB):
        out_ref[j] = mixed[j * TQ:(j + 1) * TQ]


def _out_body(x_ref, w1_ref, w2_ref, o_ref):
    xs = [x_ref[hh] for hh in range(H)]
    x0 = jnp.concatenate(xs[:H // G], axis=1).astype(BF16)
    x1 = jnp.concatenate(xs[H // G:], axis=1).astype(BF16)
    y0 = _dotd(x0, w1_ref[0])
    y1 = _dotd(x1, w1_ref[1])
    o_ref[...] = (_dotd(y0.astype(BF16), w2_ref[:INTER, :])
                  + _dotd(y1.astype(BF16), w2_ref[INTER:, :]))


def kernel(h, Wq_comp, Wq_up, Wk, Wv, qn_w, kn_w, Wg, bg, sink, Wo1, Wo2):
    h2 = h[0]
    wqcT = Wq_comp.T.astype(BF16)
    wquT = Wq_up.T.astype(BF16)
    wkT = Wk.T.astype(BF16)
    wvT = Wv.T.astype(BF16)
    kn2 = kn_w[None, :]
    qn2 = qn_w[None, :]
    wg_pad = jnp.zeros((D, 8), F32).at[:, :3].set(Wg.T)
    bg_pad = jnp.zeros((1, 8), F32).at[0, :3].set(bg)
    et_m = (jnp.arange(T)[:, None] // CR == jnp.arange(NB)[None, :]).astype(BF16)
    pos = jnp.arange(T, dtype=F32)
    inv = 1.0 / (THETA ** (jnp.arange(0, ROPE, 2, dtype=F32) / ROPE))
    fr = pos[:, None] * inv[None, :]
    cs_t, sn_t = jnp.cos(fr), jnp.sin(fr)
    avg_m = ((jnp.arange(TQ)[None, :] // CR == jnp.arange(TQ // CR)[:, None])
             .astype(F32) / CR)

    q, ke, v, kc, vc = pl.pallas_call(
        _proj_body,
        grid=(NT,),
        in_specs=[
            pl.BlockSpec((TQ, HID), lambda i: (i, 0)),
            pl.BlockSpec((HID, QCD), lambda i: (0, 0)),
            pl.BlockSpec((QCD, HID), lambda i: (0, 0)),
            pl.BlockSpec((HID, D), lambda i: (0, 0)),
            pl.BlockSpec((HID, D), lambda i: (0, 0)),
            pl.BlockSpec((1, D), lambda i: (0, 0)),
            pl.BlockSpec((TQ, HALF), lambda i: (i, 0)),
            pl.BlockSpec((TQ, HALF), lambda i: (i, 0)),
            pl.BlockSpec((TQ // CR, TQ), lambda i: (0, 0)),
            pl.BlockSpec((TQ, NB), lambda i: (i, 0)),
        ],
        out_specs=[
            pl.BlockSpec((H, TQ, D), lambda i: (0, i, 0)),
            pl.BlockSpec((TQ, 2 * D), lambda i: (i, 0)),
            pl.BlockSpec((TQ, D), lambda i: (i, 0)),
            pl.BlockSpec((TQ // CR, D), lambda i: (i, 0)),
            pl.BlockSpec((TQ // CR, D), lambda i: (i, 0)),
        ],
        out_shape=[
            jax.ShapeDtypeStruct((H, T, D), F32),
            jax.ShapeDtypeStruct((T, 2 * D), BF16),
            jax.ShapeDtypeStruct((T, D), BF16),
            jax.ShapeDtypeStruct((NB, D), F32),
            jax.ShapeDtypeStruct((NB, D), F32),
        ],
    )(h2, wqcT, wquT, wkT, wvT, kn2, cs_t, sn_t, avg_m, et_m)

    mixed = pl.pallas_call(
        _attn_body,
        grid=(H // HB, NT),
        in_specs=[
            pl.BlockSpec((HB, TQ, D), lambda hh, i: (hh, i, 0)),
            pl.BlockSpec((T, 2 * D), lambda hh, i: (0, 0)),
            pl.BlockSpec((T, D), lambda hh, i: (0, 0)),
            pl.BlockSpec((NB, D), lambda hh, i: (0, 0)),
            pl.BlockSpec((NB, D), lambda hh, i: (0, 0)),
            pl.BlockSpec((D, 8), lambda hh, i: (0, 0)),
            pl.BlockSpec((1, 8), lambda hh, i: (0, 0)),
            pl.BlockSpec((1, D), lambda hh, i: (0, 0)),
            pl.BlockSpec((TQ, HALF), lambda hh, i: (i, 0)),
            pl.BlockSpec((TQ, HALF), lambda hh, i: (i, 0)),
            pl.BlockSpec(memory_space=pltpu.SMEM),
        ],
        out_specs=pl.BlockSpec((HB, TQ, D), lambda hh, i: (hh, i, 0)),
        out_shape=jax.ShapeDtypeStruct((H, T, D), F32),
    )(q, ke, v, kc, vc, wg_pad, bg_pad, qn2, cs_t, sn_t, sink)

    w1t = jnp.transpose(Wo1, (0, 2, 1)).astype(BF16)
    w2t = Wo2.T.astype(BF16)
    y = pl.pallas_call(
        _out_body,
        grid=(NT,),
        in_specs=[
            pl.BlockSpec((H, TQ, D), lambda i: (0, i, 0)),
            pl.BlockSpec((G, QCD, INTER), lambda i: (0, 0, 0)),
            pl.BlockSpec((G * INTER, HID), lambda i: (0, 0)),
        ],
        out_specs=pl.BlockSpec((TQ, HID), lambda i: (i, 0)),
        out_shape=jax.ShapeDtypeStruct((T, HID), F32),
    )(mixed, w1t, w2t)
    return y[None]
